# Initial kernel scaffold; baseline (speedup 1.0000x reference)
import jax
import jax.numpy as jnp
from jax.experimental import pallas as pl


def kernel(token_ids, embedding):
    raise NotImplementedError("write your pallas kernel here")


# speedup vs baseline: 1.6828x; 1.6828x over previous
import functools

import jax
import jax.numpy as jnp
from jax import lax
from jax.experimental import pallas as pl
from jax.experimental.pallas import tpu as pltpu
from jax.experimental.pallas import tpu_sc as plsc

NUM_EMBEDDINGS = 1000000
EMBEDDING_DIM = 64
BATCH = 16384
HIST = 50

_TOTAL = BATCH * HIST
_CHUNK = 128


def _make_gather(num_workers: int, num_cores: int):
    chunks_per_w = _TOTAL // (num_workers * _CHUNK)
    per_w = chunks_per_w * _CHUNK
    mesh = plsc.VectorSubcoreMesh(core_axis_name="c", subcore_axis_name="s")

    @functools.partial(
        pl.kernel,
        mesh=mesh,
        out_type=jax.ShapeDtypeStruct((_TOTAL, EMBEDDING_DIM), jnp.float32),
        scratch_types=[
            pltpu.VMEM((chunks_per_w, _CHUNK), jnp.int32),
            pltpu.VMEM((_CHUNK, EMBEDDING_DIM), jnp.float32),
            pltpu.SemaphoreType.DMA,
        ],
        compiler_params=pltpu.CompilerParams(use_tc_tiling_on_sc=False),
    )
    def gather_kernel(idx_hbm, table_hbm, out_hbm, idx_v, rows_v, sem):
        wid = lax.axis_index("s") * num_cores + lax.axis_index("c")
        base = wid * per_w
        pltpu.sync_copy(idx_hbm.at[wid], idx_v)

        def body(j, carry):
            pltpu.async_copy(table_hbm.at[idx_v.at[j]], rows_v, sem).wait()
            pltpu.sync_copy(rows_v, out_hbm.at[pl.ds(base + j * _CHUNK, _CHUNK)])
            return carry

        lax.fori_loop(0, chunks_per_w, body, 0)

    return gather_kernel


def kernel(token_ids, embedding):
    info = plsc.get_sparse_core_info()
    num_workers = info.num_cores * info.num_subcores
    idx = token_ids.reshape(num_workers, -1, _CHUNK).astype(jnp.int32)
    out = _make_gather(num_workers, info.num_cores)(idx, embedding)
    return out.reshape(BATCH, HIST, EMBEDDING_DIM)


# speedup vs baseline: 1.8756x; 1.1146x over previous
import functools

import jax
import jax.numpy as jnp
from jax import lax
from jax.experimental import pallas as pl
from jax.experimental.pallas import tpu as pltpu
from jax.experimental.pallas import tpu_sc as plsc

NUM_EMBEDDINGS = 1000000
EMBEDDING_DIM = 64
BATCH = 16384
HIST = 50

_TOTAL = BATCH * HIST
_CHUNK = 128
_K = 4
_NBUF = 2


def _make_gather(num_workers: int, num_cores: int):
    chunks_per_w = _TOTAL // (num_workers * _CHUNK)
    per_w = chunks_per_w * _CHUNK
    groups = chunks_per_w // _K
    outer = groups // _NBUF
    mesh = plsc.VectorSubcoreMesh(core_axis_name="c", subcore_axis_name="s")

    @functools.partial(
        pl.kernel,
        mesh=mesh,
        out_type=jax.ShapeDtypeStruct((_TOTAL, EMBEDDING_DIM), jnp.float32),
        scratch_types=[
            pltpu.VMEM((chunks_per_w, _CHUNK), jnp.int32),
            pltpu.VMEM((_NBUF, _K * _CHUNK, EMBEDDING_DIM), jnp.float32),
            [pltpu.SemaphoreType.DMA] * _NBUF,
        ],
        compiler_params=pltpu.CompilerParams(use_tc_tiling_on_sc=False),
    )
    def gather_kernel(idx_hbm, table_hbm, out_hbm, idx_v, rows_v, sems):
        wid = lax.axis_index("s") * num_cores + lax.axis_index("c")
        base = wid * per_w
        pltpu.sync_copy(idx_hbm.at[wid], idx_v)

        def fire(g, b):
            for k in range(_K):
                pltpu.async_copy(
                    table_hbm.at[idx_v.at[g * _K + k]],
                    rows_v.at[b, pl.ds(k * _CHUNK, _CHUNK)],
                    sems[b],
                )

        def drain(g, b):
            for k in range(_K):
                pltpu.make_async_copy(
                    table_hbm.at[idx_v.at[g * _K + k]],
                    rows_v.at[b, pl.ds(k * _CHUNK, _CHUNK)],
                    sems[b],
                ).wait()

        for b in range(_NBUF):
            fire(b, b)

        def body(g2, carry):
            for b in range(_NBUF):
                g = g2 * _NBUF + b
                drain(g, b)
                pltpu.sync_copy(
                    rows_v.at[b],
                    out_hbm.at[pl.ds(base + g * _K * _CHUNK, _K * _CHUNK)],
                )

                @pl.when(g2 + 1 < outer)
                def _():
                    fire(g + _NBUF, b)

            return carry

        lax.fori_loop(0, outer, body, 0)

    return gather_kernel


def kernel(token_ids, embedding):
    info = plsc.get_sparse_core_info()
    num_workers = info.num_cores * info.num_subcores
    idx = token_ids.reshape(num_workers, -1, _CHUNK).astype(jnp.int32)
    out = _make_gather(num_workers, info.num_cores)(idx, embedding)
    return out.reshape(BATCH, HIST, EMBEDDING_DIM)
